# Initial kernel scaffold; baseline (speedup 1.0000x reference)
import jax
import jax.numpy as jnp
from jax.experimental import pallas as pl


def kernel(x, edge_index, W1_rel, b1_rel, W1_root, W2_rel, b2_rel, W2_root, W3_rel, b3_rel, W3_root):
    raise NotImplementedError("write your pallas kernel here")


# speedup vs baseline: 4.5876x; 4.5876x over previous
import functools

import jax
import jax.numpy as jnp
from jax import lax
from jax.experimental import pallas as pl
from jax.experimental.pallas import tpu as pltpu
from jax.experimental.pallas import tpu_sc as plsc

N_NODES = 10000
D_FEAT = 128
N_EDGES = 320000

NC = 2
NS = 16
NW = NC * NS


def _seg_body(n, d, e, chunk, rchunk,
              src_hbm, dst_hbm, x_hbm, out_hbm,
              sidx_v, didx_v, rows_v, stage_v, agg_sh, sem):
    ept = e // NW
    nchunk = ept // chunk
    nrch = n // rchunk
    riters = -(-nrch // NS)

    c = lax.axis_index("c")
    s = lax.axis_index("s")
    wid = s * NC + c

    @pl.loop(0, rchunk)
    def _zbuf(i):
        for j in range(d // 16):
            stage_v[i, pl.ds(j * 16, 16)] = jnp.zeros((16,), jnp.float32)

    @pl.loop(0, riters)
    def _zagg(i):
        k = s + i * NS

        @pl.when(k < nrch)
        def _():
            pltpu.sync_copy(stage_v, agg_sh.at[pl.ds(k * rchunk, rchunk)])

    plsc.subcore_barrier()

    @pl.loop(0, nchunk)
    def _edges(i):
        base = wid * ept + i * chunk
        pltpu.sync_copy(src_hbm.at[pl.ds(base, chunk)], sidx_v)
        pltpu.sync_copy(dst_hbm.at[pl.ds(base, chunk)], didx_v)
        pltpu.async_copy(x_hbm.at[sidx_v], rows_v, sem).wait()
        pltpu.sync_copy(rows_v, agg_sh.at[didx_v], add=True)

    plsc.subcore_barrier()

    @pl.loop(0, riters)
    def _wout(i):
        k = s + i * NS

        @pl.when(k < nrch)
        def _():
            r0 = k * rchunk
            pltpu.sync_copy(agg_sh.at[pl.ds(r0, rchunk)], stage_v)
            pltpu.sync_copy(stage_v, out_hbm.at[c, pl.ds(r0, rchunk)])


@functools.lru_cache(maxsize=None)
def _build_seg(n, d, e, chunk, rchunk, interpret=False):
    mesh = plsc.VectorSubcoreMesh(
        core_axis_name="c", subcore_axis_name="s",
        num_cores=NC, num_subcores=NS)
    return pl.kernel(
        functools.partial(_seg_body, n, d, e, chunk, rchunk),
        out_type=jax.ShapeDtypeStruct((NC, n, d), jnp.float32),
        mesh=mesh,
        scratch_types=[
            pltpu.VMEM((chunk,), jnp.int32),
            pltpu.VMEM((chunk,), jnp.int32),
            pltpu.VMEM((chunk, d), jnp.float32),
            pltpu.VMEM((rchunk, d), jnp.float32),
            pltpu.VMEM_SHARED((n, d), jnp.float32),
            pltpu.SemaphoreType.DMA,
        ],
        interpret=interpret,
    )


def _tc_body(agg_ref, h_ref, wr_ref, b_ref, wt_ref, o_ref):
    a = agg_ref[0] + agg_ref[1]
    y = jnp.dot(a, wr_ref[...], preferred_element_type=jnp.float32)
    y = y + jnp.dot(h_ref[...], wt_ref[...], preferred_element_type=jnp.float32)
    o_ref[...] = jnp.maximum(y + b_ref[...], 0.0)


@functools.lru_cache(maxsize=None)
def _build_tc(n, d, blk, interpret=False):
    return pl.pallas_call(
        _tc_body,
        grid=(n // blk,),
        in_specs=[
            pl.BlockSpec((NC, blk, d), lambda i: (0, i, 0)),
            pl.BlockSpec((blk, d), lambda i: (i, 0)),
            pl.BlockSpec((d, d), lambda i: (0, 0)),
            pl.BlockSpec((1, d), lambda i: (0, 0)),
            pl.BlockSpec((d, d), lambda i: (0, 0)),
        ],
        out_specs=pl.BlockSpec((blk, d), lambda i: (i, 0)),
        out_shape=jax.ShapeDtypeStruct((n, d), jnp.float32),
        interpret=interpret,
    )


def kernel(x, edge_index, W1_rel, b1_rel, W1_root, W2_rel, b2_rel, W2_root,
           W3_rel, b3_rel, W3_root):
    ei = edge_index.astype(jnp.int32)
    src, dst = ei[0], ei[1]
    seg = _build_seg(N_NODES, D_FEAT, N_EDGES, 80, 40)
    tc = _build_tc(N_NODES, D_FEAT, 1000)
    h = x
    for wr, b, wt in ((W1_rel, b1_rel, W1_root),
                      (W2_rel, b2_rel, W2_root),
                      (W3_rel, b3_rel, W3_root)):
        agg2 = seg(src, dst, h)
        h = tc(agg2, h, wr, b.reshape(1, D_FEAT), wt)
    return h.reshape(1, N_NODES, D_FEAT)


# speedup vs baseline: 9.1041x; 1.9845x over previous
import functools

import jax
import jax.numpy as jnp
from jax import lax
from jax.experimental import pallas as pl
from jax.experimental.pallas import tpu as pltpu
from jax.experimental.pallas import tpu_sc as plsc

N_NODES = 10000
N_PAD = 10240
D_FEAT = 128
N_EDGES = 320000

NC = 2
NS = 16
NW = NC * NS


def _seg_body(n, npad, d, e, chunk, rchunk,
              src_hbm, dst_hbm, x_hbm, out_hbm,
              sidx_v, didx_v, rows_v, stage_v, agg_sh,
              gsem0, gsem1, ssem, isem0, isem1):
    del n
    ept = e // NW
    nchunk = ept // chunk
    assert nchunk % 2 == 1 and nchunk >= 3
    rpt = npad // NS
    nrc = rpt // rchunk

    c = lax.axis_index("c")
    s = lax.axis_index("s")
    wid = s * NC + c
    gsems = (gsem0, gsem1)
    isems = (isem0, isem1)
    ebase = wid * ept

    def _fire_idx(j, b):
        pltpu.async_copy(
            src_hbm.at[pl.ds(ebase + j * chunk, chunk)], sidx_v.at[b],
            isems[b])
        pltpu.async_copy(
            dst_hbm.at[pl.ds(ebase + j * chunk, chunk)], didx_v.at[b],
            isems[b])

    def _wait_idx(j, b):
        pltpu.make_async_copy(
            src_hbm.at[pl.ds(ebase + j * chunk, chunk)], sidx_v.at[b],
            isems[b]).wait()
        pltpu.make_async_copy(
            dst_hbm.at[pl.ds(ebase + j * chunk, chunk)], didx_v.at[b],
            isems[b]).wait()

    def _fire_gather(b):
        pltpu.async_copy(x_hbm.at[sidx_v.at[b]], rows_v.at[b], gsems[b])

    def _wait_gather(b):
        pltpu.make_async_copy(
            x_hbm.at[sidx_v.at[b]], rows_v.at[b], gsems[b]).wait()

    def _fire_scat(b):
        pltpu.async_copy(rows_v.at[b], agg_sh.at[didx_v.at[b]], ssem,
                         add=True)

    def _drain_scat(b):
        pltpu.make_async_copy(
            rows_v.at[b], agg_sh.at[didx_v.at[b]], ssem).wait()

    @pl.loop(0, rchunk)
    def _zbuf(i):
        for j in range(d // 16):
            stage_v[i, pl.ds(j * 16, 16)] = jnp.zeros((16,), jnp.float32)

    @pl.loop(0, nrc)
    def _zagg(i):
        pltpu.sync_copy(stage_v,
                        agg_sh.at[pl.ds(s * rpt + i * rchunk, rchunk)])

    plsc.subcore_barrier()

    _fire_idx(0, 0)
    _wait_idx(0, 0)
    _fire_gather(0)
    _fire_idx(1, 1)
    _wait_idx(1, 1)
    _fire_gather(1)
    _wait_gather(0)
    _fire_scat(0)
    _drain_scat(0)
    _fire_idx(2, 0)
    _wait_idx(2, 0)
    _fire_gather(0)
    _wait_gather(1)
    _fire_scat(1)

    @pl.loop(2, nchunk - 1, step=2)
    def _edges(i):
        for b in range(2):
            j = i + b
            o = 1 - b
            _drain_scat(o)
            _fire_idx(j + 1, o)
            _wait_idx(j + 1, o)
            _fire_gather(o)
            _wait_gather(b)
            _fire_scat(b)

    _drain_scat(1)
    _wait_gather(0)
    _fire_scat(0)
    _drain_scat(0)

    plsc.subcore_barrier()

    @pl.loop(0, nrc)
    def _wout(i):
        r0 = s * rpt + i * rchunk
        pltpu.sync_copy(agg_sh.at[pl.ds(r0, rchunk)], stage_v)
        pltpu.sync_copy(stage_v, out_hbm.at[c, pl.ds(r0, rchunk)])


@functools.lru_cache(maxsize=None)
def _build_seg(n, npad, d, e, chunk, rchunk, interpret=False):
    mesh = plsc.VectorSubcoreMesh(
        core_axis_name="c", subcore_axis_name="s",
        num_cores=NC, num_subcores=NS)
    return pl.kernel(
        functools.partial(_seg_body, n, npad, d, e, chunk, rchunk),
        out_type=jax.ShapeDtypeStruct((NC, npad, d), jnp.float32),
        mesh=mesh,
        scratch_types=[
            pltpu.VMEM((2, chunk), jnp.int32),
            pltpu.VMEM((2, chunk), jnp.int32),
            pltpu.VMEM((2, chunk, d), jnp.float32),
            pltpu.VMEM((rchunk, d), jnp.float32),
            pltpu.VMEM_SHARED((npad, d), jnp.float32),
            pltpu.SemaphoreType.DMA,
            pltpu.SemaphoreType.DMA,
            pltpu.SemaphoreType.DMA,
            pltpu.SemaphoreType.DMA,
            pltpu.SemaphoreType.DMA,
        ],
        interpret=interpret,
    )


def _tc_body(agg_ref, h_ref, wr_ref, b_ref, wt_ref, o_ref):
    a = agg_ref[0] + agg_ref[1]
    y = jnp.dot(a, wr_ref[...], preferred_element_type=jnp.float32)
    y = y + jnp.dot(h_ref[...], wt_ref[...], preferred_element_type=jnp.float32)
    o_ref[...] = jnp.maximum(y + b_ref[...], 0.0)


@functools.lru_cache(maxsize=None)
def _build_tc(n, d, blk, interpret=False):
    return pl.pallas_call(
        _tc_body,
        grid=(n // blk,),
        in_specs=[
            pl.BlockSpec((NC, blk, d), lambda i: (0, i, 0)),
            pl.BlockSpec((blk, d), lambda i: (i, 0)),
            pl.BlockSpec((d, d), lambda i: (0, 0)),
            pl.BlockSpec((1, d), lambda i: (0, 0)),
            pl.BlockSpec((d, d), lambda i: (0, 0)),
        ],
        out_specs=pl.BlockSpec((blk, d), lambda i: (i, 0)),
        out_shape=jax.ShapeDtypeStruct((n, d), jnp.float32),
        interpret=interpret,
    )


def kernel(x, edge_index, W1_rel, b1_rel, W1_root, W2_rel, b2_rel, W2_root,
           W3_rel, b3_rel, W3_root):
    ei = edge_index.astype(jnp.int32)
    src, dst = ei[0], ei[1]
    seg = _build_seg(N_NODES, N_PAD, D_FEAT, N_EDGES, 80, 40)
    tc = _build_tc(N_NODES, D_FEAT, 1000)
    h = x
    for wr, b, wt in ((W1_rel, b1_rel, W1_root),
                      (W2_rel, b2_rel, W2_root),
                      (W3_rel, b3_rel, W3_root)):
        agg2 = seg(src, dst, h)
        h = tc(agg2, h, wr, b.reshape(1, D_FEAT), wt)
    return h.reshape(1, N_NODES, D_FEAT)


# speedup vs baseline: 12.9197x; 1.4191x over previous
import functools

import jax
import jax.numpy as jnp
from jax import lax
from jax.experimental import pallas as pl
from jax.experimental.pallas import tpu as pltpu
from jax.experimental.pallas import tpu_sc as plsc

N_NODES = 10000
N_PAD = 10240
D_FEAT = 128
N_EDGES = 320000

NC = 2
NS = 16
NW = NC * NS


def _seg_body(npad, d, epad, chunk, rchunk,
              src_hbm, dst_hbm, x_hbm, out_hbm,
              sidx_v, didx_v, rows_v, agg_sh,
              gsem0, gsem1, gsem2, ssem,
              isem0, isem1, isem2, isem3, isem4, isem5):
    ept = epad // NW
    nchunk = ept // chunk
    assert nchunk % 6 == 2 and nchunk >= 14
    rpt = npad // NS
    nrc = rpt // rchunk
    assert rpt % rchunk == 0 and rchunk <= chunk

    c = lax.axis_index("c")
    s = lax.axis_index("s")
    wid = s * NC + c
    gsems = (gsem0, gsem1, gsem2)
    isems = (isem0, isem1, isem2, isem3, isem4, isem5)
    ebase = wid * ept

    def _fire_idx(j, m):
        pltpu.async_copy(
            src_hbm.at[pl.ds(ebase + j * chunk, chunk)], sidx_v.at[m],
            isems[m])
        pltpu.async_copy(
            dst_hbm.at[pl.ds(ebase + j * chunk, chunk)], didx_v.at[m],
            isems[m])

    def _wait_idx(j, m):
        pltpu.make_async_copy(
            src_hbm.at[pl.ds(ebase + j * chunk, chunk)], sidx_v.at[m],
            isems[m]).wait()
        pltpu.make_async_copy(
            dst_hbm.at[pl.ds(ebase + j * chunk, chunk)], didx_v.at[m],
            isems[m]).wait()

    def _fire_g(mi, b):
        pltpu.async_copy(x_hbm.at[sidx_v.at[mi]], rows_v.at[b], gsems[b])

    def _wait_g(mi, b):
        pltpu.make_async_copy(
            x_hbm.at[sidx_v.at[mi]], rows_v.at[b], gsems[b]).wait()

    def _fire_s(mi, b):
        pltpu.async_copy(rows_v.at[b], agg_sh.at[didx_v.at[mi]], ssem,
                         add=True)

    def _drain_s(mi, b):
        pltpu.make_async_copy(
            rows_v.at[b], agg_sh.at[didx_v.at[mi]], ssem).wait()

    @pl.loop(0, rchunk)
    def _zbuf(i):
        for j in range(d // 16):
            rows_v[0, i, pl.ds(j * 16, 16)] = jnp.zeros((16,), jnp.float32)

    @pl.loop(0, nrc)
    def _zagg(i):
        pltpu.sync_copy(rows_v.at[0].at[pl.ds(0, rchunk)],
                        agg_sh.at[pl.ds(s * rpt + i * rchunk, rchunk)])

    plsc.subcore_barrier()

    def _step(j, m, drain=True, fidx=True, fg=True):
        if drain:
            _drain_s((m - 2) % 6, (m - 2) % 3)
        if fidx:
            _fire_idx(j + 3, (m + 3) % 6)
        if fg:
            _wait_idx(j + 1, (m + 1) % 6)
            _fire_g((m + 1) % 6, (m + 1) % 3)
        _wait_g(m % 6, m % 3)
        _fire_s(m % 6, m % 3)

    _fire_idx(0, 0)
    _fire_idx(1, 1)
    _fire_idx(2, 2)
    _wait_idx(0, 0)
    _fire_g(0, 0)
    for j in range(5):
        _step(j, j, drain=(j >= 2))

    @pl.loop(5, nchunk - 3, step=6)
    def _edges(i):
        for b in range(6):
            _step(i + b, 5 + b)

    _step(nchunk - 3, nchunk - 3, fidx=False)
    _step(nchunk - 2, nchunk - 2, fidx=False)
    _step(nchunk - 1, nchunk - 1, fidx=False, fg=False)
    _drain_s((nchunk - 2) % 6, (nchunk - 2) % 3)
    _drain_s((nchunk - 1) % 6, (nchunk - 1) % 3)

    plsc.subcore_barrier()

    @pl.loop(0, nrc)
    def _wout(i):
        r0 = s * rpt + i * rchunk
        stage = rows_v.at[0].at[pl.ds(0, rchunk)]
        pltpu.sync_copy(agg_sh.at[pl.ds(r0, rchunk)], stage)
        pltpu.sync_copy(stage, out_hbm.at[c, pl.ds(r0, rchunk)])


@functools.lru_cache(maxsize=None)
def _build_seg(npad, d, epad, chunk, rchunk, interpret=False):
    mesh = plsc.VectorSubcoreMesh(
        core_axis_name="c", subcore_axis_name="s",
        num_cores=NC, num_subcores=NS)
    return pl.kernel(
        functools.partial(_seg_body, npad, d, epad, chunk, rchunk),
        out_type=jax.ShapeDtypeStruct((NC, npad, d), jnp.float32),
        mesh=mesh,
        scratch_types=[
            pltpu.VMEM((6, chunk), jnp.int32),
            pltpu.VMEM((6, chunk), jnp.int32),
            pltpu.VMEM((3, chunk, d), jnp.float32),
            pltpu.VMEM_SHARED((npad, d), jnp.float32),
        ] + [pltpu.SemaphoreType.DMA] * 10,
        interpret=interpret,
    )


def _tc_body(agg_ref, h_ref, wr_ref, b_ref, wt_ref, o_ref):
    a = agg_ref[0] + agg_ref[1]
    y = jnp.dot(a, wr_ref[...], preferred_element_type=jnp.float32)
    y = y + jnp.dot(h_ref[...], wt_ref[...], preferred_element_type=jnp.float32)
    o_ref[...] = jnp.maximum(y + b_ref[...], 0.0)


@functools.lru_cache(maxsize=None)
def _build_tc(n, d, blk, interpret=False):
    return pl.pallas_call(
        _tc_body,
        grid=(n // blk,),
        in_specs=[
            pl.BlockSpec((NC, blk, d), lambda i: (0, i, 0)),
            pl.BlockSpec((blk, d), lambda i: (i, 0)),
            pl.BlockSpec((d, d), lambda i: (0, 0)),
            pl.BlockSpec((1, d), lambda i: (0, 0)),
            pl.BlockSpec((d, d), lambda i: (0, 0)),
        ],
        out_specs=pl.BlockSpec((blk, d), lambda i: (i, 0)),
        out_shape=jax.ShapeDtypeStruct((n, d), jnp.float32),
        interpret=interpret,
    )


def kernel(x, edge_index, W1_rel, b1_rel, W1_root, W2_rel, b2_rel, W2_root,
           W3_rel, b3_rel, W3_root):
    chunk, nchunk = 112, 92
    epad = NW * nchunk * chunk
    npages = N_PAD - N_NODES
    p = epad - N_EDGES
    ei = edge_index.astype(jnp.int32)
    pad_src = (jnp.arange(p, dtype=jnp.int32) * 7) % N_NODES
    pad_dst = N_NODES + jnp.arange(p, dtype=jnp.int32) % npages
    src = jnp.concatenate([ei[0], pad_src])
    dst = jnp.concatenate([ei[1], pad_dst])
    seg = _build_seg(N_PAD, D_FEAT, epad, chunk, 80)
    tc = _build_tc(N_NODES, D_FEAT, 1000)
    h = x
    for wr, b, wt in ((W1_rel, b1_rel, W1_root),
                      (W2_rel, b2_rel, W2_root),
                      (W3_rel, b3_rel, W3_root)):
        agg2 = seg(src, dst, h)
        h = tc(agg2, h, wr, b.reshape(1, D_FEAT), wt)
    return h.reshape(1, N_NODES, D_FEAT)


# speedup vs baseline: 13.2636x; 1.0266x over previous
import functools

import jax
import jax.numpy as jnp
from jax import lax
from jax.experimental import pallas as pl
from jax.experimental.pallas import tpu as pltpu
from jax.experimental.pallas import tpu_sc as plsc

N_NODES = 10000
N_PAD = 10240
D_FEAT = 128
N_EDGES = 320000

NC = 2
NS = 16
NW = NC * NS


def _seg_body(npad, d, epad, chunk, rchunk,
              src_hbm, dst_hbm, x_hbm, out_hbm,
              sidx_v, didx_v, rows_v, agg_sh,
              gsem0, gsem1, gsem2, ssem,
              isem0, isem1, isem2, isem3, isem4, isem5):
    ept = epad // NW
    nchunk = ept // chunk
    assert nchunk % 6 == 2 and nchunk >= 14
    rpt = npad // NS
    nrc = rpt // rchunk
    assert rpt % rchunk == 0 and rchunk <= chunk

    c = lax.axis_index("c")
    s = lax.axis_index("s")
    wid = s * NC + c
    gsems = (gsem0, gsem1, gsem2)
    isems = (isem0, isem1, isem2, isem3, isem4, isem5)
    ebase = wid * ept

    def _fire_idx(j, m):
        pltpu.async_copy(
            src_hbm.at[pl.ds(ebase + j * chunk, chunk)], sidx_v.at[m],
            isems[m])
        pltpu.async_copy(
            dst_hbm.at[pl.ds(ebase + j * chunk, chunk)], didx_v.at[m],
            isems[m])

    def _wait_idx(j, m):
        pltpu.make_async_copy(
            src_hbm.at[pl.ds(ebase + j * chunk, chunk)], sidx_v.at[m],
            isems[m]).wait()
        pltpu.make_async_copy(
            dst_hbm.at[pl.ds(ebase + j * chunk, chunk)], didx_v.at[m],
            isems[m]).wait()

    def _fire_g(mi, b):
        pltpu.async_copy(x_hbm.at[sidx_v.at[mi]], rows_v.at[b], gsems[b])

    def _wait_g(mi, b):
        pltpu.make_async_copy(
            x_hbm.at[sidx_v.at[mi]], rows_v.at[b], gsems[b]).wait()

    def _fire_s(mi, b):
        pltpu.async_copy(rows_v.at[b], agg_sh.at[didx_v.at[mi]], ssem,
                         add=True)

    def _drain_s(mi, b):
        pltpu.make_async_copy(
            rows_v.at[b], agg_sh.at[didx_v.at[mi]], ssem).wait()

    _fire_idx(0, 0)
    _fire_idx(1, 1)
    _fire_idx(2, 2)

    @pl.loop(0, rchunk)
    def _zbuf(i):
        for j in range(d // 16):
            rows_v[0, i, pl.ds(j * 16, 16)] = jnp.zeros((16,), jnp.float32)

    @pl.loop(0, nrc)
    def _zagg(i):
        pltpu.sync_copy(rows_v.at[0].at[pl.ds(0, rchunk)],
                        agg_sh.at[pl.ds(s * rpt + i * rchunk, rchunk)])

    plsc.subcore_barrier()

    def _step(j, m, drain=True, fidx=True, fg=True):
        if drain:
            _drain_s((m - 2) % 6, (m - 2) % 3)
        if fidx:
            _fire_idx(j + 3, (m + 3) % 6)
        if fg:
            _wait_idx(j + 1, (m + 1) % 6)
            _fire_g((m + 1) % 6, (m + 1) % 3)
        _wait_g(m % 6, m % 3)
        _fire_s(m % 6, m % 3)

    _wait_idx(0, 0)
    _fire_g(0, 0)
    for j in range(5):
        _step(j, j, drain=(j >= 2))

    @pl.loop(5, nchunk - 3, step=6)
    def _edges(i):
        for b in range(6):
            _step(i + b, 5 + b)

    _step(nchunk - 3, nchunk - 3, fidx=False)
    _step(nchunk - 2, nchunk - 2, fidx=False)
    _step(nchunk - 1, nchunk - 1, fidx=False, fg=False)
    _drain_s((nchunk - 2) % 6, (nchunk - 2) % 3)
    _drain_s((nchunk - 1) % 6, (nchunk - 1) % 3)

    plsc.subcore_barrier()

    assert nrc % 2 == 0 and nrc >= 4

    def _wchunk(k, b, wait_prev):
        r0 = s * rpt + k * rchunk
        stage = rows_v.at[b].at[pl.ds(0, rchunk)]
        if wait_prev:
            rp = s * rpt + (k - 2) * rchunk
            pltpu.make_async_copy(
                rows_v.at[b].at[pl.ds(0, rchunk)],
                out_hbm.at[c, pl.ds(rp, rchunk)], gsems[b]).wait()
        pltpu.sync_copy(agg_sh.at[pl.ds(r0, rchunk)], stage)
        pltpu.async_copy(stage, out_hbm.at[c, pl.ds(r0, rchunk)], gsems[b])

    _wchunk(0, 0, False)
    _wchunk(1, 1, False)

    @pl.loop(2, nrc, step=2)
    def _wout(i):
        for b in range(2):
            _wchunk(i + b, b, True)

    pltpu.make_async_copy(
        rows_v.at[0].at[pl.ds(0, rchunk)],
        out_hbm.at[c, pl.ds(s * rpt + (nrc - 2) * rchunk, rchunk)],
        gsems[0]).wait()
    pltpu.make_async_copy(
        rows_v.at[1].at[pl.ds(0, rchunk)],
        out_hbm.at[c, pl.ds(s * rpt + (nrc - 1) * rchunk, rchunk)],
        gsems[1]).wait()


@functools.lru_cache(maxsize=None)
def _build_seg(npad, d, epad, chunk, rchunk, interpret=False):
    mesh = plsc.VectorSubcoreMesh(
        core_axis_name="c", subcore_axis_name="s",
        num_cores=NC, num_subcores=NS)
    return pl.kernel(
        functools.partial(_seg_body, npad, d, epad, chunk, rchunk),
        out_type=jax.ShapeDtypeStruct((NC, npad, d), jnp.float32),
        mesh=mesh,
        scratch_types=[
            pltpu.VMEM((6, chunk), jnp.int32),
            pltpu.VMEM((6, chunk), jnp.int32),
            pltpu.VMEM((3, chunk, d), jnp.float32),
            pltpu.VMEM_SHARED((npad, d), jnp.float32),
        ] + [pltpu.SemaphoreType.DMA] * 10,
        interpret=interpret,
    )


def _tc_body(agg_ref, h_ref, wr_ref, b_ref, wt_ref, o_ref):
    a = agg_ref[0] + agg_ref[1]
    y = jnp.dot(a, wr_ref[...], preferred_element_type=jnp.float32)
    y = y + jnp.dot(h_ref[...], wt_ref[...], preferred_element_type=jnp.float32)
    o_ref[...] = jnp.maximum(y + b_ref[...], 0.0)


@functools.lru_cache(maxsize=None)
def _build_tc(n, d, blk, interpret=False):
    return pl.pallas_call(
        _tc_body,
        grid=(n // blk,),
        in_specs=[
            pl.BlockSpec((NC, blk, d), lambda i: (0, i, 0)),
            pl.BlockSpec((blk, d), lambda i: (i, 0)),
            pl.BlockSpec((d, d), lambda i: (0, 0)),
            pl.BlockSpec((1, d), lambda i: (0, 0)),
            pl.BlockSpec((d, d), lambda i: (0, 0)),
        ],
        out_specs=pl.BlockSpec((blk, d), lambda i: (i, 0)),
        out_shape=jax.ShapeDtypeStruct((n, d), jnp.float32),
        interpret=interpret,
    )


def kernel(x, edge_index, W1_rel, b1_rel, W1_root, W2_rel, b2_rel, W2_root,
           W3_rel, b3_rel, W3_root):
    chunk, nchunk = 112, 92
    epad = NW * nchunk * chunk
    npages = N_PAD - N_NODES
    p = epad - N_EDGES
    ei = edge_index.astype(jnp.int32)
    pad_src = (jnp.arange(p, dtype=jnp.int32) * 7) % N_NODES
    pad_dst = N_NODES + jnp.arange(p, dtype=jnp.int32) % npages
    src = jnp.concatenate([ei[0], pad_src])
    dst = jnp.concatenate([ei[1], pad_dst])
    seg = _build_seg(N_PAD, D_FEAT, epad, chunk, 80)
    tc = _build_tc(N_NODES, D_FEAT, 1000)
    h = x
    for wr, b, wt in ((W1_rel, b1_rel, W1_root),
                      (W2_rel, b2_rel, W2_root),
                      (W3_rel, b3_rel, W3_root)):
        agg2 = seg(src, dst, h)
        h = tc(agg2, h, wr, b.reshape(1, D_FEAT), wt)
    return h.reshape(1, N_NODES, D_FEAT)


# speedup vs baseline: 13.8229x; 1.0422x over previous
import functools

import jax
import jax.numpy as jnp
from jax import lax
from jax.experimental import pallas as pl
from jax.experimental.pallas import tpu as pltpu
from jax.experimental.pallas import tpu_sc as plsc

N_NODES = 10000
N_PAD = 10240
D_FEAT = 128
N_EDGES = 320000

NC = 2
NS = 16
NW = NC * NS


def _seg_body(npad, d, epad, chunk, rchunk,
              src_hbm, dst_hbm, x_hbm, out_hbm,
              sidx_v, didx_v, rows_v, agg_sh,
              gsem0, gsem1, gsem2, gsem3, ssem,
              isem0, isem1, isem2, isem3, isem4, isem5):
    ept = epad // NW
    nchunk = ept // chunk
    assert nchunk % 12 == 5 and nchunk >= 29
    rpt = npad // NS
    nrc = rpt // rchunk
    assert rpt % rchunk == 0 and rchunk <= chunk

    c = lax.axis_index("c")
    s = lax.axis_index("s")
    wid = s * NC + c
    gsems = (gsem0, gsem1, gsem2, gsem3)
    isems = (isem0, isem1, isem2, isem3, isem4, isem5)
    ebase = wid * ept

    def _fire_idx(j, m):
        pltpu.async_copy(
            src_hbm.at[pl.ds(ebase + j * chunk, chunk)], sidx_v.at[m],
            isems[m])
        pltpu.async_copy(
            dst_hbm.at[pl.ds(ebase + j * chunk, chunk)], didx_v.at[m],
            isems[m])

    def _wait_idx(j, m):
        pltpu.make_async_copy(
            src_hbm.at[pl.ds(ebase + j * chunk, chunk)], sidx_v.at[m],
            isems[m]).wait()
        pltpu.make_async_copy(
            dst_hbm.at[pl.ds(ebase + j * chunk, chunk)], didx_v.at[m],
            isems[m]).wait()

    def _fire_g(mi, b):
        pltpu.async_copy(x_hbm.at[sidx_v.at[mi]], rows_v.at[b], gsems[b])

    def _wait_g(mi, b):
        pltpu.make_async_copy(
            x_hbm.at[sidx_v.at[mi]], rows_v.at[b], gsems[b]).wait()

    def _fire_s(mi, b):
        pltpu.async_copy(rows_v.at[b], agg_sh.at[didx_v.at[mi]], ssem,
                         add=True)

    def _drain_s(mi, b):
        pltpu.make_async_copy(
            rows_v.at[b], agg_sh.at[didx_v.at[mi]], ssem).wait()

    _fire_idx(0, 0)
    _fire_idx(1, 1)
    _fire_idx(2, 2)
    _fire_idx(3, 3)

    @pl.loop(0, rchunk)
    def _zbuf(i):
        for j in range(d // 16):
            rows_v[0, i, pl.ds(j * 16, 16)] = jnp.zeros((16,), jnp.float32)

    @pl.loop(0, nrc)
    def _zagg(i):
        pltpu.sync_copy(rows_v.at[0].at[pl.ds(0, rchunk)],
                        agg_sh.at[pl.ds(s * rpt + i * rchunk, rchunk)])

    plsc.subcore_barrier()

    def _step(j, m, drain=True, fidx=True, fg=True):
        if drain:
            _drain_s((m - 2) % 6, (m - 2) % 4)
        if fidx:
            _fire_idx(j + 4, (m + 4) % 6)
        if fg:
            _wait_idx(j + 2, (m + 2) % 6)
            _fire_g((m + 2) % 6, (m + 2) % 4)
        _wait_g(m % 6, m % 4)
        _fire_s(m % 6, m % 4)

    _wait_idx(0, 0)
    _fire_g(0, 0)
    _wait_idx(1, 1)
    _fire_g(1, 1)
    for j in range(13):
        _step(j, j, drain=(j >= 2))

    @pl.loop(13, nchunk - 4, step=12)
    def _edges(i):
        for b in range(12):
            _step(i + b, 13 + b)

    _step(nchunk - 4, nchunk - 4, fidx=False)
    _step(nchunk - 3, nchunk - 3, fidx=False)
    _step(nchunk - 2, nchunk - 2, fidx=False, fg=False)
    _step(nchunk - 1, nchunk - 1, fidx=False, fg=False)
    _drain_s((nchunk - 2) % 6, (nchunk - 2) % 4)
    _drain_s((nchunk - 1) % 6, (nchunk - 1) % 4)

    plsc.subcore_barrier()

    assert nrc % 2 == 0 and nrc >= 4

    def _wchunk(k, b, wait_prev):
        r0 = s * rpt + k * rchunk
        stage = rows_v.at[b].at[pl.ds(0, rchunk)]
        if wait_prev:
            rp = s * rpt + (k - 2) * rchunk
            pltpu.make_async_copy(
                rows_v.at[b].at[pl.ds(0, rchunk)],
                out_hbm.at[c, pl.ds(rp, rchunk)], gsems[b]).wait()
        pltpu.sync_copy(agg_sh.at[pl.ds(r0, rchunk)], stage)
        pltpu.async_copy(stage, out_hbm.at[c, pl.ds(r0, rchunk)], gsems[b])

    _wchunk(0, 0, False)
    _wchunk(1, 1, False)

    @pl.loop(2, nrc, step=2)
    def _wout(i):
        for b in range(2):
            _wchunk(i + b, b, True)

    pltpu.make_async_copy(
        rows_v.at[0].at[pl.ds(0, rchunk)],
        out_hbm.at[c, pl.ds(s * rpt + (nrc - 2) * rchunk, rchunk)],
        gsems[0]).wait()
    pltpu.make_async_copy(
        rows_v.at[1].at[pl.ds(0, rchunk)],
        out_hbm.at[c, pl.ds(s * rpt + (nrc - 1) * rchunk, rchunk)],
        gsems[1]).wait()


@functools.lru_cache(maxsize=None)
def _build_seg(npad, d, epad, chunk, rchunk, interpret=False):
    mesh = plsc.VectorSubcoreMesh(
        core_axis_name="c", subcore_axis_name="s",
        num_cores=NC, num_subcores=NS)
    return pl.kernel(
        functools.partial(_seg_body, npad, d, epad, chunk, rchunk),
        out_type=jax.ShapeDtypeStruct((NC, npad, d), jnp.float32),
        mesh=mesh,
        scratch_types=[
            pltpu.VMEM((6, chunk), jnp.int32),
            pltpu.VMEM((6, chunk), jnp.int32),
            pltpu.VMEM((4, chunk, d), jnp.float32),
            pltpu.VMEM_SHARED((npad, d), jnp.float32),
        ] + [pltpu.SemaphoreType.DMA] * 11,
        interpret=interpret,
    )


def _tc_body(agg_ref, h_ref, wr_ref, b_ref, wt_ref, o_ref):
    a = agg_ref[0] + agg_ref[1]
    y = jnp.dot(a, wr_ref[...], preferred_element_type=jnp.float32)
    y = y + jnp.dot(h_ref[...], wt_ref[...], preferred_element_type=jnp.float32)
    o_ref[...] = jnp.maximum(y + b_ref[...], 0.0)


@functools.lru_cache(maxsize=None)
def _build_tc(n, d, blk, interpret=False):
    return pl.pallas_call(
        _tc_body,
        grid=(n // blk,),
        in_specs=[
            pl.BlockSpec((NC, blk, d), lambda i: (0, i, 0)),
            pl.BlockSpec((blk, d), lambda i: (i, 0)),
            pl.BlockSpec((d, d), lambda i: (0, 0)),
            pl.BlockSpec((1, d), lambda i: (0, 0)),
            pl.BlockSpec((d, d), lambda i: (0, 0)),
        ],
        out_specs=pl.BlockSpec((blk, d), lambda i: (i, 0)),
        out_shape=jax.ShapeDtypeStruct((n, d), jnp.float32),
        interpret=interpret,
    )


def kernel(x, edge_index, W1_rel, b1_rel, W1_root, W2_rel, b2_rel, W2_root,
           W3_rel, b3_rel, W3_root):
    chunk = 80
    ei = edge_index.astype(jnp.int32)
    src, dst = ei[0], ei[1]
    seg = _build_seg(N_PAD, D_FEAT, N_EDGES, chunk, 80)
    tc = _build_tc(N_NODES, D_FEAT, 1000)
    h = x
    for wr, b, wt in ((W1_rel, b1_rel, W1_root),
                      (W2_rel, b2_rel, W2_root),
                      (W3_rel, b3_rel, W3_root)):
        agg2 = seg(src, dst, h)
        h = tc(agg2, h, wr, b.reshape(1, D_FEAT), wt)
    return h.reshape(1, N_NODES, D_FEAT)
